# baseline (speedup 1.0000x reference)
import dataclasses
import functools

import jax
import jax.numpy as jnp
from jax import lax
from jax.experimental import pallas as pl
from jax.experimental.pallas import tpu as pltpu
from jax.experimental.pallas import tpu_sc as plsc

_NUM_CLASSES = 100000
_HIDDEN = 128
_B = 16384
_NC, _NS, _L = 2, 16, 16
_NW = _NC * _NS
_BPW = _B // _NW
_CH = 32
_NCH = _BPW // _CH


def kernel(labels, force_drop_ids, embedding_table):
    mesh = plsc.VectorSubcoreMesh(core_axis_name="c", subcore_axis_name="s")
    cp = pltpu.CompilerParams()
    if "needs_layout_passes" in pltpu.CompilerParams.__dataclass_fields__:
        cp = dataclasses.replace(cp, needs_layout_passes=False)

    @functools.partial(
        pl.kernel,
        mesh=mesh,
        compiler_params=cp,
        out_type=jax.ShapeDtypeStruct((_B, _HIDDEN), jnp.float32),
        scratch_types=[
            pltpu.VMEM((_BPW,), jnp.int32),
            pltpu.VMEM((_BPW,), jnp.int32),
            pltpu.VMEM((_BPW,), jnp.int32),
            pltpu.VMEM((_BPW,), jnp.int32),
            pltpu.VMEM((_NCH, _CH), jnp.int32),
            pltpu.VMEM((_NCH, _CH), jnp.int32),
            pltpu.VMEM((_CH, _HIDDEN), jnp.float32),
            pltpu.VMEM((_BPW, _HIDDEN), jnp.float32),
            pltpu.SemaphoreType.DMA,
            pltpu.SemaphoreType.DMA,
        ],
    )
    def emb_kernel(table_hbm, labels_hbm, drop_hbm, out_hbm,
                   lab_v, drop_v, cidx_f, cpos_f, idx2d, pos2d,
                   null_rep, rows_v, gsem, ssem):
        wid = lax.axis_index("s") * _NC + lax.axis_index("c")
        base = wid * _BPW
        in_cps = [
            pltpu.async_copy(labels_hbm.at[pl.ds(base, _BPW)], lab_v, gsem),
            pltpu.async_copy(drop_hbm.at[pl.ds(base, _BPW)], drop_v, gsem),
            pltpu.async_copy(table_hbm.at[pl.ds(_NUM_CLASSES, 1)],
                             null_rep.at[pl.ds(0, 1)], gsem),
        ]
        for cp_ in in_cps:
            cp_.wait()

        for h in range(0, _HIDDEN, _L):
            val = null_rep[0, pl.ds(h, _L)]
            for r in range(1, _CH):
                null_rep[r, pl.ds(h, _L)] = val

        lane = lax.iota(jnp.int32, _L)
        one = jnp.full((_L,), 1, jnp.int32)

        nulls = jnp.full((_L,), _NUM_CLASSES, jnp.int32)
        for c in range(0, _BPW, _L):
            cidx_f[pl.ds(c, _L)] = nulls

        koff = jnp.int32(0)
        for c in range(0, _BPW, _L):
            lab = lab_v[pl.ds(c, _L)]
            keep = drop_v[pl.ds(c, _L)] == 0
            pos = lane + (base + c)
            slot = koff + plsc.cumsum(one, mask=keep) - 1
            plsc.store_scatter(cidx_f, [slot], lab, mask=keep)
            plsc.store_scatter(cpos_f, [slot], pos, mask=keep)
            koff = koff + jnp.max(plsc.all_reduce_population_count(keep))
        n_kept = koff

        n_g = (n_kept + (_CH - 1)) // _CH
        for j in range(_NCH):
            for c2 in range(0, _CH, _L):
                idx2d[j, pl.ds(c2, _L)] = cidx_f[pl.ds(j * _CH + c2, _L)]

        def gather_issue(j, _):
            pltpu.async_copy(table_hbm.at[idx2d.at[j]],
                             rows_v.at[pl.ds(j * _CH, _CH)], gsem)
            return _
        lax.fori_loop(0, n_g, gather_issue, 0)

        doff = n_kept
        for c in range(0, _BPW, _L):
            drop = drop_v[pl.ds(c, _L)] != 0
            pos = lane + (base + c)
            slot = doff + plsc.cumsum(one, mask=drop) - 1
            plsc.store_scatter(cpos_f, [slot], pos, mask=drop)
            doff = doff + jnp.max(plsc.all_reduce_population_count(drop))

        for j in range(_NCH):
            for c2 in range(0, _CH, _L):
                pos2d[j, pl.ds(c2, _L)] = cpos_f[pl.ds(j * _CH + c2, _L)]

        def null_issue(j, _):
            pltpu.async_copy(null_rep, out_hbm.at[pos2d.at[j]], ssem)
            return _
        lax.fori_loop(n_g, _NCH, null_issue, 0)

        def gather_drain(j, _):
            pltpu.make_async_copy(table_hbm.at[pl.ds(0, _CH)],
                                  rows_v.at[pl.ds(0, _CH)], gsem).wait()
            return _
        lax.fori_loop(0, n_g, gather_drain, 0)

        def row_issue(j, _):
            pltpu.async_copy(rows_v.at[pl.ds(j * _CH, _CH)],
                             out_hbm.at[pos2d.at[j]], ssem)
            return _
        lax.fori_loop(0, n_g, row_issue, 0)

        def scatter_drain(j, _):
            pltpu.make_async_copy(rows_v.at[pl.ds(0, _CH)],
                                  out_hbm.at[pos2d.at[0]], ssem).wait()
            return _
        lax.fori_loop(0, _NCH, scatter_drain, 0)

    return emb_kernel(embedding_table, labels, force_drop_ids)


# speedup vs baseline: 1.2689x; 1.2689x over previous
import dataclasses
import functools

import jax
import jax.numpy as jnp
from jax import lax
from jax.experimental import pallas as pl
from jax.experimental.pallas import tpu as pltpu
from jax.experimental.pallas import tpu_sc as plsc

_NUM_CLASSES = 100000
_HIDDEN = 128
_B = 16384
_NC, _NS, _L = 2, 16, 16
_NW = _NC * _NS
_BPW = _B // _NW
_CH = 16
_NCH = _BPW // _CH
_GCH = 8


def kernel(labels, force_drop_ids, embedding_table):
    mesh = plsc.VectorSubcoreMesh(core_axis_name="c", subcore_axis_name="s")
    cp = pltpu.CompilerParams()
    if "needs_layout_passes" in pltpu.CompilerParams.__dataclass_fields__:
        cp = dataclasses.replace(cp, needs_layout_passes=False)

    @functools.partial(
        pl.kernel,
        mesh=mesh,
        compiler_params=cp,
        out_type=jax.ShapeDtypeStruct((_B, _HIDDEN), jnp.float32),
        scratch_types=[
            pltpu.VMEM((_BPW,), jnp.int32),
            pltpu.VMEM((_BPW,), jnp.int32),
            pltpu.VMEM((_BPW,), jnp.int32),
            pltpu.VMEM((_BPW,), jnp.int32),
            pltpu.VMEM((_NCH, _CH), jnp.int32),
            pltpu.VMEM((_CH, _HIDDEN), jnp.float32),
            pltpu.VMEM((_BPW, _HIDDEN), jnp.float32),
            pltpu.SemaphoreType.DMA,
            pltpu.SemaphoreType.DMA,
        ],
    )
    def emb_kernel(table_hbm, labels_hbm, drop_hbm, out_hbm,
                   lab_v, drop_v, cidx_f, cpos_f, pos2d,
                   null_rep, rows_v, gsem, ssem):
        wid = lax.axis_index("s") * _NC + lax.axis_index("c")
        base = wid * _BPW
        in_cps = [
            pltpu.async_copy(labels_hbm.at[pl.ds(base, _BPW)], lab_v, gsem),
            pltpu.async_copy(drop_hbm.at[pl.ds(base, _BPW)], drop_v, gsem),
            pltpu.async_copy(table_hbm.at[pl.ds(_NUM_CLASSES, 1)],
                             null_rep.at[pl.ds(0, 1)], gsem),
        ]
        for cp_ in in_cps:
            cp_.wait()

        for h in range(0, _HIDDEN, _L):
            val = null_rep[0, pl.ds(h, _L)]
            for r in range(1, _CH):
                null_rep[r, pl.ds(h, _L)] = val

        lane = lax.iota(jnp.int32, _L)
        one = jnp.full((_L,), 1, jnp.int32)

        nulls = jnp.full((_L,), _NUM_CLASSES, jnp.int32)
        for c in range(0, _BPW, _L):
            cidx_f[pl.ds(c, _L)] = nulls

        koff = jnp.int32(0)
        for c in range(0, _BPW, _L):
            lab = lab_v[pl.ds(c, _L)]
            keep = drop_v[pl.ds(c, _L)] == 0
            pos = lane + (base + c)
            slot = koff + plsc.cumsum(one, mask=keep) - 1
            plsc.store_scatter(cidx_f, [slot], lab, mask=keep)
            plsc.store_scatter(cpos_f, [slot], pos, mask=keep)
            koff = koff + jnp.max(plsc.all_reduce_population_count(keep))
        n_kept = koff

        n_g = (n_kept + (_CH - 1)) // _CH
        n_g8 = n_g * (_CH // _GCH)

        def gather_issue(j, _):
            pltpu.async_copy(table_hbm.at[cidx_f.at[pl.ds(j * _GCH, _GCH)]],
                             rows_v.at[pl.ds(j * _GCH, _GCH)], gsem)
            return _
        lax.fori_loop(0, n_g8, gather_issue, 0)

        doff = n_kept
        for c in range(0, _BPW, _L):
            drop = drop_v[pl.ds(c, _L)] != 0
            pos = lane + (base + c)
            slot = doff + plsc.cumsum(one, mask=drop) - 1
            plsc.store_scatter(cpos_f, [slot], pos, mask=drop)
            doff = doff + jnp.max(plsc.all_reduce_population_count(drop))

        for j in range(_NCH):
            pos2d[j, pl.ds(0, _CH)] = cpos_f[pl.ds(j * _CH, _CH)]

        def null_issue(j, _):
            pltpu.async_copy(null_rep, out_hbm.at[pos2d.at[j]], ssem)
            return _
        lax.fori_loop(n_g, _NCH, null_issue, 0)

        def gather_drain(j, _):
            pltpu.make_async_copy(table_hbm.at[pl.ds(0, _GCH)],
                                  rows_v.at[pl.ds(0, _GCH)], gsem).wait()
            return _
        lax.fori_loop(0, n_g8, gather_drain, 0)

        def row_issue(j, _):
            pltpu.async_copy(rows_v.at[pl.ds(j * _CH, _CH)],
                             out_hbm.at[pos2d.at[j]], ssem)
            return _
        lax.fori_loop(0, n_g, row_issue, 0)

        def scatter_drain(j, _):
            pltpu.make_async_copy(rows_v.at[pl.ds(0, _CH)],
                                  out_hbm.at[pos2d.at[0]], ssem).wait()
            return _
        lax.fori_loop(0, _NCH, scatter_drain, 0)

    return emb_kernel(embedding_table, labels, force_drop_ids)


# speedup vs baseline: 1.2981x; 1.0231x over previous
import dataclasses
import functools

import jax
import jax.numpy as jnp
from jax import lax
from jax.experimental import pallas as pl
from jax.experimental.pallas import tpu as pltpu
from jax.experimental.pallas import tpu_sc as plsc

_NUM_CLASSES = 100000
_HIDDEN = 128
_B = 16384
_NC, _NS, _L = 2, 16, 16
_NW = _NC * _NS
_BPW = _B // _NW
_CH = 16
_NCH = _BPW // _CH
_GCH = 8


def kernel(labels, force_drop_ids, embedding_table):
    mesh = plsc.VectorSubcoreMesh(core_axis_name="c", subcore_axis_name="s")
    cp = pltpu.CompilerParams()
    if "needs_layout_passes" in pltpu.CompilerParams.__dataclass_fields__:
        cp = dataclasses.replace(cp, needs_layout_passes=False)

    @functools.partial(
        pl.kernel,
        mesh=mesh,
        compiler_params=cp,
        out_type=jax.ShapeDtypeStruct((_B, _HIDDEN), jnp.float32),
        scratch_types=[
            pltpu.VMEM((_BPW,), jnp.int32),
            pltpu.VMEM((_BPW,), jnp.int32),
            pltpu.VMEM((_BPW,), jnp.int32),
            pltpu.VMEM((_BPW,), jnp.int32),
            pltpu.VMEM((_CH, _HIDDEN), jnp.float32),
            pltpu.VMEM((_BPW, _HIDDEN), jnp.float32),
            pltpu.SemaphoreType.DMA,
            pltpu.SemaphoreType.DMA,
        ],
    )
    def emb_kernel(table_hbm, labels_hbm, drop_hbm, out_hbm,
                   lab_v, drop_v, cidx_f, cpos_f,
                   null_rep, rows_v, gsem, ssem):
        wid = lax.axis_index("s") * _NC + lax.axis_index("c")
        base = wid * _BPW
        in_cps = [
            pltpu.async_copy(labels_hbm.at[pl.ds(base, _BPW)], lab_v, gsem),
            pltpu.async_copy(drop_hbm.at[pl.ds(base, _BPW)], drop_v, gsem),
            pltpu.async_copy(table_hbm.at[pl.ds(_NUM_CLASSES, 1)],
                             null_rep.at[pl.ds(0, 1)], gsem),
        ]
        for cp_ in in_cps:
            cp_.wait()

        for h in range(0, _HIDDEN, _L):
            val = null_rep[0, pl.ds(h, _L)]
            for r in range(1, _CH):
                null_rep[r, pl.ds(h, _L)] = val

        lane = lax.iota(jnp.int32, _L)
        one = jnp.full((_L,), 1, jnp.int32)

        nulls = jnp.full((_L,), _NUM_CLASSES, jnp.int32)
        for c in range(0, _BPW, _L):
            cidx_f[pl.ds(c, _L)] = nulls

        koff = jnp.int32(0)
        dback = jnp.int32(_BPW)
        for c in range(0, _BPW, _L):
            lab = lab_v[pl.ds(c, _L)]
            drp = drop_v[pl.ds(c, _L)]
            keep = drp == 0
            drop = drp != 0
            pos = lane + (base + c)
            kslot = koff + plsc.cumsum(one, mask=keep) - 1
            plsc.store_scatter(cidx_f, [kslot], lab, mask=keep)
            plsc.store_scatter(cpos_f, [kslot], pos, mask=keep)
            dslot = dback - plsc.cumsum(one, mask=drop)
            plsc.store_scatter(cpos_f, [dslot], pos, mask=drop)
            nk = jnp.max(plsc.all_reduce_population_count(keep))
            koff = koff + nk
            dback = dback - (_L - nk)
        n_kept = koff

        n_g = (n_kept + (_CH - 1)) // _CH
        n_g8 = n_g * (_CH // _GCH)

        def gather_issue(j, _):
            pltpu.async_copy(table_hbm.at[cidx_f.at[pl.ds(j * _GCH, _GCH)]],
                             rows_v.at[pl.ds(j * _GCH, _GCH)], gsem)
            return _
        lax.fori_loop(0, n_g8, gather_issue, 0)

        def null_issue(j, _):
            pltpu.async_copy(null_rep, out_hbm.at[cpos_f.at[pl.ds(j * _CH, _CH)]], ssem)
            return _
        lax.fori_loop(n_g, _NCH, null_issue, 0)

        def gather_drain(j, _):
            pltpu.make_async_copy(table_hbm.at[pl.ds(0, _GCH)],
                                  rows_v.at[pl.ds(0, _GCH)], gsem).wait()
            return _
        lax.fori_loop(0, n_g8, gather_drain, 0)

        def row_issue(j, _):
            pltpu.async_copy(rows_v.at[pl.ds(j * _CH, _CH)],
                             out_hbm.at[cpos_f.at[pl.ds(j * _CH, _CH)]], ssem)
            return _
        lax.fori_loop(0, n_g, row_issue, 0)

        def scatter_drain(j, _):
            pltpu.make_async_copy(rows_v.at[pl.ds(0, _CH)],
                                  out_hbm.at[cpos_f.at[pl.ds(0, _CH)]],
                                  ssem).wait()
            return _
        lax.fori_loop(0, _NCH, scatter_drain, 0)

    return emb_kernel(embedding_table, labels, force_drop_ids)


# speedup vs baseline: 1.4158x; 1.0906x over previous
import dataclasses
import functools

import jax
import jax.numpy as jnp
from jax import lax
from jax.experimental import pallas as pl
from jax.experimental.pallas import tpu as pltpu
from jax.experimental.pallas import tpu_sc as plsc

_NUM_CLASSES = 100000
_HIDDEN = 128
_B = 16384
_NC, _NS, _L = 2, 16, 16
_NW = _NC * _NS
_BPW = _B // _NW
_CH = 16
_NCH = _BPW // _CH
_GCH = 8


def kernel(labels, force_drop_ids, embedding_table):
    mesh = plsc.VectorSubcoreMesh(core_axis_name="c", subcore_axis_name="s")
    cp = pltpu.CompilerParams()
    if "needs_layout_passes" in pltpu.CompilerParams.__dataclass_fields__:
        cp = dataclasses.replace(cp, needs_layout_passes=False)

    @functools.partial(
        pl.kernel,
        mesh=mesh,
        compiler_params=cp,
        out_type=jax.ShapeDtypeStruct((_B, _HIDDEN), jnp.float32),
        scratch_types=[
            pltpu.VMEM((_BPW,), jnp.int32),
            pltpu.VMEM((_BPW,), jnp.int32),
            pltpu.VMEM((_BPW,), jnp.int32),
            pltpu.VMEM((_BPW,), jnp.int32),
            pltpu.VMEM((_CH, _HIDDEN), jnp.float32),
            pltpu.VMEM((_BPW, _HIDDEN), jnp.float32),
            pltpu.SemaphoreType.DMA,
            pltpu.SemaphoreType.DMA,
        ],
    )
    def emb_kernel(table_hbm, labels_hbm, drop_hbm, out_hbm,
                   lab_v, drop_v, cidx_f, cpos_f,
                   null_rep, rows_v, gsem, ssem):
        wid = lax.axis_index("s") * _NC + lax.axis_index("c")
        base = wid * _BPW
        in_cps = [
            pltpu.async_copy(labels_hbm.at[pl.ds(base, _BPW)], lab_v, gsem),
            pltpu.async_copy(drop_hbm.at[pl.ds(base, _BPW)], drop_v, gsem),
            pltpu.async_copy(table_hbm.at[pl.ds(_NUM_CLASSES, 1)],
                             null_rep.at[pl.ds(0, 1)], gsem),
        ]
        for cp_ in in_cps:
            cp_.wait()

        for h in range(0, _HIDDEN, _L):
            val = null_rep[0, pl.ds(h, _L)]
            for r in range(1, _CH):
                null_rep[r, pl.ds(h, _L)] = val

        lane = lax.iota(jnp.int32, _L)
        one = jnp.full((_L,), 1, jnp.int32)

        nulls = jnp.full((_L,), _NUM_CLASSES, jnp.int32)
        for c in range(0, _BPW, _L):
            cidx_f[pl.ds(c, _L)] = nulls

        koff = jnp.int32(0)
        dback = jnp.int32(_BPW)
        for c in range(0, _BPW, _L):
            lab = lab_v[pl.ds(c, _L)]
            drp = drop_v[pl.ds(c, _L)]
            keep = drp == 0
            drop = drp != 0
            pos = lane + (base + c)
            kslot = koff + plsc.cumsum(one, mask=keep) - 1
            plsc.store_scatter(cidx_f, [kslot], lab, mask=keep)
            plsc.store_scatter(cpos_f, [kslot], pos, mask=keep)
            dslot = dback - plsc.cumsum(one, mask=drop)
            plsc.store_scatter(cpos_f, [dslot], pos, mask=drop)
            nk = jnp.max(plsc.all_reduce_population_count(keep))
            koff = koff + nk
            dback = dback - (_L - nk)
        n_kept = koff

        n_g = (n_kept + (_CH - 1)) // _CH
        n_g8 = n_g * (_CH // _GCH)

        def gather_issue(j, _):
            pltpu.async_copy(table_hbm.at[cidx_f.at[pl.ds(j * _GCH, _GCH)]],
                             rows_v.at[pl.ds(j * _GCH, _GCH)], gsem)
            return _
        lax.fori_loop(0, n_g8, gather_issue, 0)

        def null_issue(j, _):
            pltpu.async_copy(null_rep, out_hbm.at[cpos_f.at[pl.ds(j * _CH, _CH)]], ssem)
            return _
        lax.fori_loop(n_g, _NCH, null_issue, 0)

        def drain_and_scatter(j, _):
            pltpu.make_async_copy(table_hbm.at[pl.ds(0, _GCH)],
                                  rows_v.at[pl.ds(0, _GCH)], gsem).wait()
            pltpu.async_copy(rows_v.at[pl.ds(j * _GCH, _GCH)],
                             out_hbm.at[cpos_f.at[pl.ds(j * _GCH, _GCH)]],
                             ssem)
            return _
        lax.fori_loop(0, n_g8, drain_and_scatter, 0)

        def row_drain(j, _):
            pltpu.make_async_copy(rows_v.at[pl.ds(0, _GCH)],
                                  out_hbm.at[cpos_f.at[pl.ds(0, _GCH)]],
                                  ssem).wait()
            return _
        lax.fori_loop(0, n_g8, row_drain, 0)

        def null_drain(j, _):
            pltpu.make_async_copy(null_rep,
                                  out_hbm.at[cpos_f.at[pl.ds(0, _CH)]],
                                  ssem).wait()
            return _
        lax.fori_loop(n_g, _NCH, null_drain, 0)

    return emb_kernel(embedding_table, labels, force_drop_ids)
